# Initial kernel scaffold; baseline (speedup 1.0000x reference)
import jax
import jax.numpy as jnp
from jax.experimental import pallas as pl


def kernel(x, edge_index, convW, convb, bn_g, bn_b, linW, linb):
    raise NotImplementedError("write your pallas kernel here")


# speedup vs baseline: 6.2264x; 6.2264x over previous
import functools

import jax
import jax.numpy as jnp
from jax import lax
from jax.experimental import pallas as pl
from jax.experimental.pallas import tpu as pltpu
from jax.experimental.pallas import tpu_sc as plsc

N = 10000
E = 320000
HID = 128
OUT = 64
L = 5

NC = 2
NS = 16
NW = NC * NS
CHUNK = 128
NCHUNK = 2560
CPT = NCHUNK // NW
EPAD = NCHUNK * CHUNK
NPAD = 10240
STRIPE = NPAD // NS


def _worker(c, s):
    return s * NC + c


@functools.cache
def _make_deg_sc():
    mesh = plsc.VectorSubcoreMesh(core_axis_name="c", subcore_axis_name="s")
    return pl.kernel(
        _deg_sc_body,
        out_type=jax.ShapeDtypeStruct((NC, NPAD, HID), jnp.float32),
        mesh=mesh,
        scratch_types=[
            pltpu.VMEM((CPT, CHUNK), jnp.int32),
            pltpu.VMEM((CHUNK, HID), jnp.float32),
            pltpu.VMEM_SHARED((NPAD, HID), jnp.float32),
        ],
    )


def _deg_sc_body(dst_hbm, out_hbm, didx, buf, acc):
    c = lax.axis_index("c")
    s = lax.axis_index("s")
    w = _worker(c, s)

    def _fill(val):
        def body(i, carry):
            for jj in range(HID // 16):
                buf[i, pl.ds(jj * 16, 16)] = jnp.full((16,), val, jnp.float32)
            return carry
        lax.fori_loop(0, CHUNK, body, 0)

    _fill(0.0)
    for k in range(STRIPE // CHUNK):
        pltpu.sync_copy(buf, acc.at[pl.ds(s * STRIPE + k * CHUNK, CHUNK)])
    _fill(1.0)
    pltpu.sync_copy(dst_hbm.at[pl.ds(w * CPT, CPT)], didx)
    plsc.subcore_barrier()

    def chunk(j, carry):
        pltpu.sync_copy(buf, acc.at[didx.at[j]], add=True)
        return carry
    lax.fori_loop(0, CPT, chunk, 0)
    plsc.subcore_barrier()

    pltpu.sync_copy(acc.at[pl.ds(s * STRIPE, STRIPE)],
                    out_hbm.at[c, pl.ds(s * STRIPE, STRIPE)])


@functools.cache
def _make_scatter_sc():
    mesh = plsc.VectorSubcoreMesh(core_axis_name="c", subcore_axis_name="s")
    return pl.kernel(
        _scatter_sc_body,
        out_type=jax.ShapeDtypeStruct((NC, NPAD, HID), jnp.float32),
        mesh=mesh,
        scratch_types=[
            pltpu.VMEM((CPT, CHUNK), jnp.int32),
            pltpu.VMEM((CPT, CHUNK), jnp.int32),
            pltpu.VMEM((CHUNK, HID), jnp.float32),
            pltpu.VMEM_SHARED((NPAD, HID), jnp.float32),
            pltpu.SemaphoreType.DMA,
        ],
    )


def _scatter_sc_body(m_hbm, src_hbm, dst_hbm, out_hbm, sidx, didx, rows, acc,
                     sem):
    c = lax.axis_index("c")
    s = lax.axis_index("s")
    w = _worker(c, s)

    def zbody(i, carry):
        for jj in range(HID // 16):
            rows[i, pl.ds(jj * 16, 16)] = jnp.zeros((16,), jnp.float32)
        return carry
    lax.fori_loop(0, CHUNK, zbody, 0)
    for k in range(STRIPE // CHUNK):
        pltpu.sync_copy(rows, acc.at[pl.ds(s * STRIPE + k * CHUNK, CHUNK)])

    pltpu.sync_copy(src_hbm.at[pl.ds(w * CPT, CPT)], sidx)
    pltpu.sync_copy(dst_hbm.at[pl.ds(w * CPT, CPT)], didx)
    plsc.subcore_barrier()

    def chunk(j, carry):
        pltpu.async_copy(m_hbm.at[sidx.at[j]], rows, sem).wait()
        pltpu.sync_copy(rows, acc.at[didx.at[j]], add=True)
        return carry
    lax.fori_loop(0, CPT, chunk, 0)
    plsc.subcore_barrier()

    pltpu.sync_copy(acc.at[pl.ds(s * STRIPE, STRIPE)],
                    out_hbm.at[c, pl.ds(s * STRIPE, STRIPE)])


def _tc0_body(degp_ref, x_ref, w_ref, dinv_ref, m_ref):
    deg = degp_ref[0, 0:N, 0:1] + degp_ref[1, 0:N, 0:1] + 1.0
    dinv = lax.rsqrt(jnp.maximum(deg, 1.0))
    dinv_ref[...] = dinv
    m_ref[...] = dinv * jnp.dot(x_ref[...], w_ref[...],
                                preferred_element_type=jnp.float32,
                                precision=lax.Precision.HIGHEST)


def _tc_mid_body(p_ref, m_ref, dinv_ref, cb_ref, g_ref, bb_ref, w_ref,
                 hmax_ref, mout_ref, hmaxout_ref):
    acc = p_ref[0, 0:N, :] + p_ref[1, 0:N, :]
    sm = dinv_ref[...] * (acc + m_ref[...]) + cb_ref[...]
    mean = jnp.mean(sm, axis=0, keepdims=True)
    var = jnp.mean((sm - mean) ** 2, axis=0, keepdims=True)
    h = (sm - mean) * lax.rsqrt(var + 1e-5) * g_ref[...] + bb_ref[...]
    h = jnp.maximum(h, 0.0)
    hmaxout_ref[...] = jnp.maximum(hmax_ref[...], h)
    mout_ref[...] = dinv_ref[...] * jnp.dot(h, w_ref[...],
                                            preferred_element_type=jnp.float32,
                                            precision=lax.Precision.HIGHEST)


def _tc_final_body(p_ref, m_ref, dinv_ref, cb_ref, g_ref, bb_ref, lw_ref,
                   lb_ref, hmax_ref, out_ref):
    acc = p_ref[0, 0:N, :] + p_ref[1, 0:N, :]
    sm = dinv_ref[...] * (acc + m_ref[...]) + cb_ref[...]
    mean = jnp.mean(sm, axis=0, keepdims=True)
    var = jnp.mean((sm - mean) ** 2, axis=0, keepdims=True)
    h = (sm - mean) * lax.rsqrt(var + 1e-5) * g_ref[...] + bb_ref[...]
    h = jnp.maximum(h, 0.0)
    hmax = jnp.maximum(hmax_ref[...], h)
    logits = jnp.dot(hmax, lw_ref[...], preferred_element_type=jnp.float32,
                     precision=lax.Precision.HIGHEST) + lb_ref[...]
    mx = jnp.max(logits, axis=-1, keepdims=True)
    sh = logits - mx
    lse = jnp.log(jnp.sum(jnp.exp(sh), axis=-1, keepdims=True))
    out_ref[...] = sh - lse


_TC_PARAMS = pltpu.CompilerParams(vmem_limit_bytes=100 * 1024 * 1024)

_tc0 = pl.pallas_call(
    _tc0_body,
    out_shape=(jax.ShapeDtypeStruct((N, 1), jnp.float32),
               jax.ShapeDtypeStruct((N, HID), jnp.float32)),
    compiler_params=_TC_PARAMS,
)

_tc_mid = pl.pallas_call(
    _tc_mid_body,
    out_shape=(jax.ShapeDtypeStruct((N, HID), jnp.float32),
               jax.ShapeDtypeStruct((N, HID), jnp.float32)),
    compiler_params=_TC_PARAMS,
)

_tc_final = pl.pallas_call(
    _tc_final_body,
    out_shape=jax.ShapeDtypeStruct((N, OUT), jnp.float32),
    compiler_params=_TC_PARAMS,
)


def kernel(x, edge_index, convW, convb, bn_g, bn_b, linW, linb):
    ei = edge_index.astype(jnp.int32)
    pad_src = jnp.zeros((EPAD - E,), jnp.int32)
    pad_dst = jnp.full((EPAD - E,), N, jnp.int32)
    src2d = jnp.concatenate([ei[0], pad_src]).reshape(NCHUNK, CHUNK)
    dst2d = jnp.concatenate([ei[1], pad_dst]).reshape(NCHUNK, CHUNK)

    degp = _make_deg_sc()(dst2d)
    dinv, m = _tc0(degp, x, convW[0])

    scatter_sc = _make_scatter_sc()
    hmax = jnp.zeros((N, HID), jnp.float32)
    for i in range(L):
        p = scatter_sc(m, src2d, dst2d)
        cb = convb[i].reshape(1, HID)
        g = bn_g[i].reshape(1, HID)
        bb = bn_b[i].reshape(1, HID)
        if i < L - 1:
            m, hmax = _tc_mid(p, m, dinv, cb, g, bb, convW[i + 1], hmax)
        else:
            out = _tc_final(p, m, dinv, cb, g, bb, linW,
                            linb.reshape(1, OUT), hmax)
    return out


# speedup vs baseline: 7.0412x; 1.1309x over previous
import functools

import jax
import jax.numpy as jnp
from jax import lax
from jax.experimental import pallas as pl
from jax.experimental.pallas import tpu as pltpu
from jax.experimental.pallas import tpu_sc as plsc

N = 10000
E = 320000
HID = 128
OUT = 64
L = 5

NC = 2
NS = 16
NW = NC * NS
CHUNK = 128
NCHUNK = 2560
CPT = NCHUNK // NW
EPAD = NCHUNK * CHUNK
NPAD = 10240
STRIPE = NPAD // NS
NBUF = 2
PHASES = 2
CPP = CPT // PHASES


def _worker(c, s):
    return s * NC + c


@functools.cache
def _make_deg_sc():
    mesh = plsc.VectorSubcoreMesh(core_axis_name="c", subcore_axis_name="s")
    return pl.kernel(
        _deg_sc_body,
        out_type=jax.ShapeDtypeStruct((NC, NPAD, HID), jnp.float32),
        mesh=mesh,
        scratch_types=[
            pltpu.VMEM((CPT, CHUNK), jnp.int32),
            pltpu.VMEM((CHUNK, HID), jnp.float32),
            pltpu.VMEM_SHARED((NPAD, HID), jnp.float32),
            pltpu.SemaphoreType.DMA,
        ],
    )


def _deg_sc_body(dst_hbm, out_hbm, didx, buf, acc, sem):
    c = lax.axis_index("c")
    s = lax.axis_index("s")
    w = _worker(c, s)

    def _fill(val):
        def body(i, carry):
            for jj in range(HID // 16):
                buf[i, pl.ds(jj * 16, 16)] = jnp.full((16,), val, jnp.float32)
            return carry
        lax.fori_loop(0, CHUNK, body, 0)

    _fill(0.0)
    for k in range(STRIPE // CHUNK):
        pltpu.sync_copy(buf, acc.at[pl.ds(s * STRIPE + k * CHUNK, CHUNK)])
    _fill(1.0)
    pltpu.sync_copy(dst_hbm.at[pl.ds(w * CPT, CPT)], didx)
    plsc.subcore_barrier()

    def chunk(j, carry):
        pltpu.sync_copy(buf, acc.at[didx.at[j]], add=True)
        return carry
    lax.fori_loop(0, CPT, chunk, 0)
    plsc.subcore_barrier()

    pltpu.sync_copy(acc.at[pl.ds(s * STRIPE, STRIPE)],
                    out_hbm.at[c, pl.ds(s * STRIPE, STRIPE)])


@functools.cache
def _make_scatter_sc():
    mesh = plsc.VectorSubcoreMesh(core_axis_name="c", subcore_axis_name="s")
    return pl.kernel(
        _scatter_sc_body,
        out_type=jax.ShapeDtypeStruct((NC, NPAD, HID), jnp.float32),
        mesh=mesh,
        scratch_types=[
            pltpu.VMEM((CPP, CHUNK), jnp.int32),
            pltpu.VMEM((CPP, CHUNK), jnp.int32),
            pltpu.VMEM((CHUNK, HID), jnp.float32),
            pltpu.VMEM((CHUNK, HID), jnp.float32),
            pltpu.VMEM_SHARED((NPAD, HID), jnp.float32),
            pltpu.SemaphoreType.DMA,
            pltpu.SemaphoreType.DMA,
        ],
    )


def _scatter_sc_body(m_hbm, src_hbm, dst_hbm, out_hbm, sidx, didx, r0, r1,
                     acc, g0, g1):
    c = lax.axis_index("c")
    s = lax.axis_index("s")
    w = _worker(c, s)
    rows = (r0, r1)
    gsem = (g0, g1)

    def zbody(i, carry):
        for jj in range(HID // 16):
            r0[i, pl.ds(jj * 16, 16)] = jnp.zeros((16,), jnp.float32)
        return carry
    lax.fori_loop(0, CHUNK, zbody, 0)
    for k in range(STRIPE // CHUNK):
        pltpu.sync_copy(r0, acc.at[pl.ds(s * STRIPE + k * CHUNK, CHUNK)])
    plsc.subcore_barrier()

    for ph in range(PHASES):
        base = w * CPT + ph * CPP
        pltpu.sync_copy(src_hbm.at[pl.ds(base, CPP)], sidx)
        pltpu.sync_copy(dst_hbm.at[pl.ds(base, CPP)], didx)
        for b in range(NBUF):
            pltpu.async_copy(m_hbm.at[sidx.at[b]], rows[b], gsem[b])

        def body(jj, carry):
            for b in range(NBUF):
                j = jj * NBUF + b
                pltpu.make_async_copy(m_hbm.at[sidx.at[j]], rows[b],
                                      gsem[b]).wait()
                pltpu.sync_copy(rows[b], acc.at[didx.at[j]], add=True)

                @pl.when(j + NBUF < CPP)
                def _():
                    pltpu.async_copy(m_hbm.at[sidx.at[j + NBUF]], rows[b],
                                     gsem[b])
            return carry
        lax.fori_loop(0, CPP // NBUF, body, 0)
    plsc.subcore_barrier()

    pltpu.sync_copy(acc.at[pl.ds(s * STRIPE, STRIPE)],
                    out_hbm.at[c, pl.ds(s * STRIPE, STRIPE)])


def _tc0_body(degp_ref, x_ref, w_ref, dinv_ref, m_ref):
    deg = degp_ref[0, 0:N, 0:1] + degp_ref[1, 0:N, 0:1] + 1.0
    dinv = lax.rsqrt(jnp.maximum(deg, 1.0))
    dinv_ref[...] = dinv
    m_ref[...] = dinv * jnp.dot(x_ref[...], w_ref[...],
                                preferred_element_type=jnp.float32,
                                precision=lax.Precision.HIGHEST)


def _tc_mid_body(p_ref, m_ref, dinv_ref, cb_ref, g_ref, bb_ref, w_ref,
                 hmax_ref, mout_ref, hmaxout_ref):
    acc = p_ref[0, 0:N, :] + p_ref[1, 0:N, :]
    sm = dinv_ref[...] * (acc + m_ref[...]) + cb_ref[...]
    mean = jnp.mean(sm, axis=0, keepdims=True)
    var = jnp.mean((sm - mean) ** 2, axis=0, keepdims=True)
    h = (sm - mean) * lax.rsqrt(var + 1e-5) * g_ref[...] + bb_ref[...]
    h = jnp.maximum(h, 0.0)
    hmaxout_ref[...] = jnp.maximum(hmax_ref[...], h)
    mout_ref[...] = dinv_ref[...] * jnp.dot(h, w_ref[...],
                                            preferred_element_type=jnp.float32,
                                            precision=lax.Precision.HIGHEST)


def _tc_final_body(p_ref, m_ref, dinv_ref, cb_ref, g_ref, bb_ref, lw_ref,
                   lb_ref, hmax_ref, out_ref):
    acc = p_ref[0, 0:N, :] + p_ref[1, 0:N, :]
    sm = dinv_ref[...] * (acc + m_ref[...]) + cb_ref[...]
    mean = jnp.mean(sm, axis=0, keepdims=True)
    var = jnp.mean((sm - mean) ** 2, axis=0, keepdims=True)
    h = (sm - mean) * lax.rsqrt(var + 1e-5) * g_ref[...] + bb_ref[...]
    h = jnp.maximum(h, 0.0)
    hmax = jnp.maximum(hmax_ref[...], h)
    logits = jnp.dot(hmax, lw_ref[...], preferred_element_type=jnp.float32,
                     precision=lax.Precision.HIGHEST) + lb_ref[...]
    mx = jnp.max(logits, axis=-1, keepdims=True)
    sh = logits - mx
    lse = jnp.log(jnp.sum(jnp.exp(sh), axis=-1, keepdims=True))
    out_ref[...] = sh - lse


_TC_PARAMS = pltpu.CompilerParams(vmem_limit_bytes=100 * 1024 * 1024)

_tc0 = pl.pallas_call(
    _tc0_body,
    out_shape=(jax.ShapeDtypeStruct((N, 1), jnp.float32),
               jax.ShapeDtypeStruct((N, HID), jnp.float32)),
    compiler_params=_TC_PARAMS,
)

_tc_mid = pl.pallas_call(
    _tc_mid_body,
    out_shape=(jax.ShapeDtypeStruct((N, HID), jnp.float32),
               jax.ShapeDtypeStruct((N, HID), jnp.float32)),
    compiler_params=_TC_PARAMS,
)

_tc_final = pl.pallas_call(
    _tc_final_body,
    out_shape=jax.ShapeDtypeStruct((N, OUT), jnp.float32),
    compiler_params=_TC_PARAMS,
)


def kernel(x, edge_index, convW, convb, bn_g, bn_b, linW, linb):
    ei = edge_index.astype(jnp.int32)
    pad_src = jnp.zeros((EPAD - E,), jnp.int32)
    pad_dst = jnp.full((EPAD - E,), N, jnp.int32)
    src2d = jnp.concatenate([ei[0], pad_src]).reshape(NCHUNK, CHUNK)
    dst2d = jnp.concatenate([ei[1], pad_dst]).reshape(NCHUNK, CHUNK)

    degp = _make_deg_sc()(dst2d)
    dinv, m = _tc0(degp, x, convW[0])

    scatter_sc = _make_scatter_sc()
    hmax = jnp.zeros((N, HID), jnp.float32)
    for i in range(L):
        p = scatter_sc(m, src2d, dst2d)
        cb = convb[i].reshape(1, HID)
        g = bn_g[i].reshape(1, HID)
        bb = bn_b[i].reshape(1, HID)
        if i < L - 1:
            m, hmax = _tc_mid(p, m, dinv, cb, g, bb, convW[i + 1], hmax)
        else:
            out = _tc_final(p, m, dinv, cb, g, bb, linW,
                            linb.reshape(1, OUT), hmax)
    return out
